# baseline (speedup 1.0000x reference)
import functools

import jax
import jax.numpy as jnp
import numpy as np
from jax import lax
from jax.experimental import pallas as pl
from jax.experimental.pallas import tpu as pltpu
import jax.experimental.pallas.tpu_sc as plsc

N_NODES = 10000
D = 128
E_EDGES = 320000
E_TOT = E_EDGES + N_NODES
NW = 32
B = 80
NG = B // 16
T_BATCH = 129
NI = T_BATCH // 3
PER_W = B * T_BATCH
E_PAD = NW * PER_W
T_ALL = E_PAD // B
N_PAD = 10240
STRIPE = N_PAD // 16


def _tc_prologue(x_ref, w_ref, as_ref, ad_ref, h_ref, asum_ref, adsum_ref):
    h = jnp.dot(x_ref[...], w_ref[...], preferred_element_type=jnp.float32)
    h_ref[...] = h
    asum_ref[...] = jnp.sum(h * as_ref[...][None, :], axis=1, keepdims=True)
    adsum_ref[...] = jnp.sum(h * ad_ref[...][None, :], axis=1, keepdims=True)


def _sc_edges(ipack_hbm, h_hbm, asrc_hbm, adst_hbm,
              u_out, s_out,
              idx_0, idx_1, idx_2, srcv_0, srcv_1, srcv_2,
              dstv_0, dstv_1, dstv_2, av_0, av_1, av_2, bv_0, bv_1, bv_2,
              p_0, p_1, p_2, rows_0, rows_1, rows_2,
              sem_i_0, sem_i_1, sem_i_2, sem_g_0, sem_g_1, sem_g_2,
              sem_s_0, sem_s_1, sem_s_2,
              u_sh, s_sh):
    c = lax.axis_index("c")
    s = lax.axis_index("s")
    wid = s * 2 + c
    row0 = wid * T_BATCH

    slot = {
        0: (idx_0, srcv_0, dstv_0, av_0, bv_0, p_0, rows_0,
            sem_i_0, sem_g_0, sem_s_0),
        1: (idx_1, srcv_1, dstv_1, av_1, bv_1, p_1, rows_1,
            sem_i_1, sem_g_1, sem_s_1),
        2: (idx_2, srcv_2, dstv_2, av_2, bv_2, p_2, rows_2,
            sem_i_2, sem_g_2, sem_s_2),
    }

    def fire_idx(t, x):
        idx, _, _, _, _, _, _, sem_i, _, _ = slot[x]
        pltpu.async_copy(ipack_hbm.at[row0 + t], idx, sem_i)

    def drain_scatter(x):
        _, _, dstv, _, _, p_v, rows, _, _, sem_s = slot[x]
        pltpu.make_async_copy(rows, u_sh.at[dstv], sem_s).wait()
        pltpu.make_async_copy(p_v, s_sh.at[dstv], sem_s).wait()

    def fire_gathers(x):
        idx, srcv, dstv, av, bv, _, rows, sem_i, sem_g, _ = slot[x]
        pltpu.make_async_copy(ipack_hbm.at[0], idx, sem_i).wait()
        for j in range(NG):
            sl = pl.ds(j * 16, 16)
            srcv[sl] = idx[sl]
            dstv[sl] = idx[pl.ds(B + j * 16, 16)]
        pltpu.async_copy(asrc_hbm.at[srcv], av, sem_g)
        pltpu.async_copy(adst_hbm.at[dstv], bv, sem_g)
        pltpu.async_copy(h_hbm.at[srcv], rows, sem_g)

    def process(t, x):
        idx, srcv, dstv, av, bv, p_v, rows, sem_i, sem_g, sem_s = slot[x]
        pltpu.make_async_copy(asrc_hbm.at[srcv], av, sem_g).wait()
        pltpu.make_async_copy(adst_hbm.at[dstv], bv, sem_g).wait()
        pltpu.make_async_copy(h_hbm.at[srcv], rows, sem_g).wait()
        base = row0 * B + t * B
        for j in range(NG):
            sl = pl.ds(j * 16, 16)
            a = av[sl] + bv[sl]
            e = jnp.where(a >= 0.0, a, 0.2 * a)
            p = jnp.exp(e)
            gid = base + j * 16 + lax.iota(jnp.int32, 16)
            p_v[sl] = jnp.where(gid < E_TOT, p, 0.0)

        @plsc.parallel_loop(0, NG, unroll=1)
        def scale_group(g):
            pg = p_v[pl.ds(g * 16, 16)]
            for l in range(16):
                pv = pg[l]
                row = g * 16 + l
                for j in range(8):
                    sl = pl.ds(j * 16, 16)
                    rows[row, sl] = rows[row, sl] * pv

        pltpu.async_copy(rows, u_sh.at[dstv], sem_s, add=True)
        pltpu.async_copy(p_v, s_sh.at[dstv], sem_s, add=True)

    def zero_rows(b, carry):
        for j in range(8):
            rows_0[b, pl.ds(j * 16, 16)] = jnp.zeros((16,), jnp.float32)
        return carry
    lax.fori_loop(0, B, zero_rows, 0)
    for k in range(STRIPE // B):
        pltpu.sync_copy(rows_0, u_sh.at[pl.ds(s * STRIPE + k * B, B)])
        pltpu.sync_copy(rows_0.at[0, pl.ds(0, B)],
                        s_sh.at[pl.ds(s * STRIPE + k * B, B)])
    plsc.subcore_barrier()

    def phase(t, x, do_s_drain, do_g_fire, do_i_fire):
        if do_s_drain:
            drain_scatter((x + 1) % 3)
        if do_g_fire:
            fire_gathers((x + 1) % 3)
        if do_i_fire:
            fire_idx(t + 3, x)
        process(t, x)

    fire_idx(0, 0)
    fire_idx(1, 1)
    fire_idx(2, 2)
    fire_gathers(0)

    phase(0, 0, False, True, True)
    phase(1, 1, False, True, True)
    phase(2, 2, True, True, True)

    def body(i, carry):
        t = 3 * i
        phase(t + 0, 0, True, True, True)
        phase(t + 1, 1, True, True, True)
        phase(t + 2, 2, True, True, True)
        return carry
    lax.fori_loop(1, NI - 1, body, 0)

    tl = T_BATCH - 3
    phase(tl + 0, 0, True, True, False)
    phase(tl + 1, 1, True, True, False)
    phase(tl + 2, 2, True, False, False)
    drain_scatter((tl + 1) % 3)
    drain_scatter((tl + 2) % 3)

    plsc.subcore_barrier()
    pltpu.sync_copy(u_sh.at[pl.ds(s * STRIPE, STRIPE)],
                    u_out.at[c, pl.ds(s * STRIPE, STRIPE)])
    pltpu.sync_copy(s_sh.at[pl.ds(s * STRIPE, STRIPE)],
                    s_out.at[c, pl.ds(s * STRIPE, STRIPE)])


def _tc_epilogue(u_ref, s_ref, b_ref, o_ref):
    u = u_ref[0, :N_NODES, :] + u_ref[1, :N_NODES, :]
    den = s_ref[0, :N_NODES, :] + s_ref[1, :N_NODES, :] + 1e-16
    o = u / den + b_ref[...][None, :]
    o_ref[...] = jnp.where(o >= 0.0, o, 0.01 * o)


@jax.jit
def _run(ipack, x, W, att_src, att_dst, bias):
    h, asum, adsum = pl.pallas_call(
        _tc_prologue,
        out_shape=[
            jax.ShapeDtypeStruct((N_NODES, D), jnp.float32),
            jax.ShapeDtypeStruct((N_NODES, 1), jnp.float32),
            jax.ShapeDtypeStruct((N_NODES, 1), jnp.float32),
        ],
    )(x, W, att_src, att_dst)

    mesh = plsc.VectorSubcoreMesh(core_axis_name="c", subcore_axis_name="s")
    sc = pl.kernel(
        _sc_edges,
        out_type=[
            jax.ShapeDtypeStruct((2, N_PAD, D), jnp.float32),
            jax.ShapeDtypeStruct((2, N_PAD), jnp.float32),
        ],
        mesh=mesh,
        scratch_types=(
            [pltpu.VMEM((2 * B,), jnp.int32) for _ in range(3)]
            + [pltpu.VMEM((B,), jnp.int32) for _ in range(3)]
            + [pltpu.VMEM((B,), jnp.int32) for _ in range(3)]
            + [pltpu.VMEM((B,), jnp.float32) for _ in range(3)]
            + [pltpu.VMEM((B,), jnp.float32) for _ in range(3)]
            + [pltpu.VMEM((B,), jnp.float32) for _ in range(3)]
            + [pltpu.VMEM((B, D), jnp.float32) for _ in range(3)]
            + [pltpu.SemaphoreType.DMA for _ in range(9)]
            + [pltpu.VMEM_SHARED((N_PAD, D), jnp.float32),
               pltpu.VMEM_SHARED((N_PAD,), jnp.float32)]
        ),
    )
    u2, s2 = sc(ipack, h, asum.reshape(-1), adsum.reshape(-1))

    out = pl.pallas_call(
        _tc_epilogue,
        out_shape=jax.ShapeDtypeStruct((N_NODES, D), jnp.float32),
    )(u2, s2.reshape(2, N_PAD, 1), bias)
    return out


_TAIL_LEN = E_PAD - E_EDGES
_tail = np.concatenate([
    np.arange(N_NODES, dtype=np.int32),
    np.arange(_TAIL_LEN - N_NODES, dtype=np.int32) % N_NODES,
]).reshape(-1, B)
_TAIL_PACK = np.concatenate([_tail, _tail], axis=1)


def kernel(edge_index, x, W, att_src, att_dst, bias):
    src_b = edge_index[0].astype(jnp.int32).reshape(E_EDGES // B, B)
    dst_b = edge_index[1].astype(jnp.int32).reshape(E_EDGES // B, B)
    epack = jnp.concatenate([src_b, dst_b], axis=1)
    ipack = jnp.concatenate([epack, _TAIL_PACK], axis=0)
    return _run(ipack, x, W, att_src, att_dst, bias)


# speedup vs baseline: 1.0535x; 1.0535x over previous
import jax
import jax.numpy as jnp
from jax import lax
from jax.experimental import pallas as pl
from jax.experimental.pallas import tpu as pltpu
import jax.experimental.pallas.tpu_sc as plsc

N_NODES = 10000
D = 128
E_EDGES = 320000
NW = 32
B = 80
NG = B // 16
T_BATCH = 125
PER_W = B * T_BATCH
N_PAD = 10240
STRIPE = N_PAD // 16


def _tc_prologue(x_ref, w_ref, as_ref, ad_ref, h_ref, asum_ref, adsum_ref):
    h = jnp.dot(x_ref[...], w_ref[...], preferred_element_type=jnp.float32)
    h_ref[...] = h
    asum_ref[...] = jnp.sum(h * as_ref[...][None, :], axis=1, keepdims=True)
    adsum_ref[...] = jnp.sum(h * ad_ref[...][None, :], axis=1, keepdims=True)


def _sc_edges(ei_hbm, h_hbm, asrc_hbm, adst_hbm,
              u_out, s_out,
              srcv_0, srcv_1, srcv_2, dstv_0, dstv_1, dstv_2,
              dsc_0, dsc_1, dsc_2,
              av_0, av_1, av_2, bv_0, bv_1, bv_2,
              p_0, p_1, p_2, rows_0, rows_1, rows_2,
              sem_i_0, sem_i_1, sem_i_2, sem_g_0, sem_g_1, sem_g_2,
              sem_s_0, sem_s_1, sem_s_2,
              u_sh, s_sh):
    c = lax.axis_index("c")
    s = lax.axis_index("s")
    wid = s * 2 + c
    row0 = wid * T_BATCH

    slot = {
        0: (srcv_0, dstv_0, dsc_0, av_0, bv_0, p_0, rows_0,
            sem_i_0, sem_g_0, sem_s_0),
        1: (srcv_1, dstv_1, dsc_1, av_1, bv_1, p_1, rows_1,
            sem_i_1, sem_g_1, sem_s_1),
        2: (srcv_2, dstv_2, dsc_2, av_2, bv_2, p_2, rows_2,
            sem_i_2, sem_g_2, sem_s_2),
    }

    def fire_idx(t, x):
        srcv, dstv, _, _, _, _, _, sem_i, _, _ = slot[x]
        off = (row0 + t) * B
        pltpu.async_copy(ei_hbm.at[pl.ds(off, B)], srcv, sem_i)
        pltpu.async_copy(ei_hbm.at[pl.ds(E_EDGES + off, B)], dstv, sem_i)

    def drain_scatter(x):
        _, _, dsc, _, _, p_v, rows, _, _, sem_s = slot[x]
        pltpu.make_async_copy(rows, u_sh.at[dsc], sem_s).wait()
        pltpu.make_async_copy(p_v, s_sh.at[dsc], sem_s).wait()

    def fire_gathers(x):
        srcv, dstv, _, av, bv, _, rows, sem_i, sem_g, _ = slot[x]
        pltpu.make_async_copy(ei_hbm.at[pl.ds(0, B)], srcv, sem_i).wait()
        pltpu.make_async_copy(ei_hbm.at[pl.ds(0, B)], dstv, sem_i).wait()
        pltpu.async_copy(asrc_hbm.at[srcv], av, sem_g)
        pltpu.async_copy(adst_hbm.at[dstv], bv, sem_g)
        pltpu.async_copy(h_hbm.at[srcv], rows, sem_g)

    def process(t, x, do_i_fire):
        srcv, dstv, dsc, av, bv, p_v, rows, sem_i, sem_g, sem_s = slot[x]
        pltpu.make_async_copy(asrc_hbm.at[srcv], av, sem_g).wait()
        pltpu.make_async_copy(adst_hbm.at[dstv], bv, sem_g).wait()
        pltpu.make_async_copy(h_hbm.at[srcv], rows, sem_g).wait()
        for j in range(NG):
            sl = pl.ds(j * 16, 16)
            dsc[sl] = dstv[sl]
        if do_i_fire:
            fire_idx(t + 3, x)
        for j in range(NG):
            sl = pl.ds(j * 16, 16)
            a = av[sl] + bv[sl]
            e = jnp.where(a >= 0.0, a, 0.2 * a)
            p_v[sl] = jnp.exp(e)

        @plsc.parallel_loop(0, NG, unroll=1)
        def scale_group(g):
            pg = p_v[pl.ds(g * 16, 16)]
            for l in range(16):
                pv = pg[l]
                row = g * 16 + l
                for j in range(8):
                    sl = pl.ds(j * 16, 16)
                    rows[row, sl] = rows[row, sl] * pv

        pltpu.async_copy(rows, u_sh.at[dsc], sem_s, add=True)
        pltpu.async_copy(p_v, s_sh.at[dsc], sem_s, add=True)

    def zero_rows(b, carry):
        for j in range(8):
            rows_0[b, pl.ds(j * 16, 16)] = jnp.zeros((16,), jnp.float32)
        return carry
    lax.fori_loop(0, B, zero_rows, 0)
    for k in range(STRIPE // B):
        pltpu.sync_copy(rows_0, u_sh.at[pl.ds(s * STRIPE + k * B, B)])
        pltpu.sync_copy(rows_0.at[0, pl.ds(0, B)],
                        s_sh.at[pl.ds(s * STRIPE + k * B, B)])
    plsc.subcore_barrier()

    def phase(t, x, do_s_drain, do_g_fire, do_i_fire):
        if do_s_drain:
            drain_scatter((x + 1) % 3)
        if do_g_fire:
            fire_gathers((x + 1) % 3)
        process(t, x, do_i_fire)

    fire_idx(0, 0)
    fire_idx(1, 1)
    fire_idx(2, 2)
    fire_gathers(0)

    phase(0, 0, False, True, True)
    phase(1, 1, False, True, True)
    phase(2, 2, True, True, True)

    def body(i, carry):
        t = 3 * i
        phase(t + 0, 0, True, True, True)
        phase(t + 1, 1, True, True, True)
        phase(t + 2, 2, True, True, True)
        return carry
    lax.fori_loop(1, (T_BATCH - 5) // 3, body, 0)

    phase(120, 0, True, True, True)
    phase(121, 1, True, True, True)
    phase(122, 2, True, True, False)
    phase(123, 0, True, True, False)
    phase(124, 1, True, False, False)
    drain_scatter(0)
    drain_scatter(1)

    plsc.subcore_barrier()
    pltpu.sync_copy(u_sh.at[pl.ds(s * STRIPE, STRIPE)],
                    u_out.at[c, pl.ds(s * STRIPE, STRIPE)])
    pltpu.sync_copy(s_sh.at[pl.ds(s * STRIPE, STRIPE)],
                    s_out.at[c, pl.ds(s * STRIPE, STRIPE)])


def _tc_epilogue(u_ref, s_ref, h_ref, as_ref, ad_ref, b_ref, o_ref):
    a = as_ref[...] + ad_ref[...]
    e = jnp.where(a >= 0.0, a, 0.2 * a)
    p_self = jnp.exp(e)
    u = u_ref[0, :N_NODES, :] + u_ref[1, :N_NODES, :] + p_self * h_ref[...]
    den = s_ref[0, :N_NODES, :] + s_ref[1, :N_NODES, :] + p_self + 1e-16
    o = u / den + b_ref[...][None, :]
    o_ref[...] = jnp.where(o >= 0.0, o, 0.01 * o)


@jax.jit
def _run(ei, x, W, att_src, att_dst, bias):
    h, asum, adsum = pl.pallas_call(
        _tc_prologue,
        out_shape=[
            jax.ShapeDtypeStruct((N_NODES, D), jnp.float32),
            jax.ShapeDtypeStruct((N_NODES, 1), jnp.float32),
            jax.ShapeDtypeStruct((N_NODES, 1), jnp.float32),
        ],
    )(x, W, att_src, att_dst)

    mesh = plsc.VectorSubcoreMesh(core_axis_name="c", subcore_axis_name="s")
    sc = pl.kernel(
        _sc_edges,
        out_type=[
            jax.ShapeDtypeStruct((2, N_PAD, D), jnp.float32),
            jax.ShapeDtypeStruct((2, N_PAD), jnp.float32),
        ],
        mesh=mesh,
        scratch_types=(
            [pltpu.VMEM((B,), jnp.int32) for _ in range(9)]
            + [pltpu.VMEM((B,), jnp.float32) for _ in range(9)]
            + [pltpu.VMEM((B, D), jnp.float32) for _ in range(3)]
            + [pltpu.SemaphoreType.DMA for _ in range(9)]
            + [pltpu.VMEM_SHARED((N_PAD, D), jnp.float32),
               pltpu.VMEM_SHARED((N_PAD,), jnp.float32)]
        ),
    )
    u2, s2 = sc(ei, h, asum.reshape(-1), adsum.reshape(-1))

    out = pl.pallas_call(
        _tc_epilogue,
        out_shape=jax.ShapeDtypeStruct((N_NODES, D), jnp.float32),
    )(u2, s2.reshape(2, N_PAD, 1), h, asum, adsum, bias)
    return out


def kernel(edge_index, x, W, att_src, att_dst, bias):
    return _run(edge_index.astype(jnp.int32).reshape(-1), x, W, att_src, att_dst, bias)


# speedup vs baseline: 1.2046x; 1.1434x over previous
import jax
import jax.numpy as jnp
from jax import lax
from jax.experimental import pallas as pl
from jax.experimental.pallas import tpu as pltpu
import jax.experimental.pallas.tpu_sc as plsc

N_NODES = 10000
D = 128
E_EDGES = 320000
NW = 32
B = 80
NG = B // 16
T_BATCH = 125
PER_W = B * T_BATCH
N_PAD = 10240
STRIPE = N_PAD // 16


def _tc_prologue(x_ref, w_ref, as_ref, ad_ref,
                 h_ref, a1_ref, a2_ref, asum_ref, adsum_ref):
    h = jnp.dot(x_ref[...], w_ref[...], preferred_element_type=jnp.float32)
    h_ref[...] = h
    asum = jnp.sum(h * as_ref[...][None, :], axis=1, keepdims=True)
    adsum = jnp.sum(h * ad_ref[...][None, :], axis=1, keepdims=True)
    asum_ref[...] = asum
    adsum_ref[...] = adsum
    a1_ref[...] = asum.reshape(-1)
    a2_ref[...] = adsum.reshape(-1)


def _sc_edges(ei_hbm, h_hbm, asrc_hbm, adst_hbm,
              u_out, s_out,
              srcv_0, srcv_1, srcv_2, srcv_3,
              dstv_0, dstv_1, dstv_2, dstv_3,
              dsc_0, dsc_1, dsc_2, dsc_3,
              av_0, av_1, av_2, av_3, bv_0, bv_1, bv_2, bv_3,
              p_0, p_1, p_2, p_3, rows_0, rows_1, rows_2, rows_3,
              sem_i_0, sem_i_1, sem_i_2, sem_i_3,
              sem_g_0, sem_g_1, sem_g_2, sem_g_3,
              sem_s_0, sem_s_1, sem_s_2, sem_s_3,
              u_sh, s_sh):
    c = lax.axis_index("c")
    s = lax.axis_index("s")
    wid = s * 2 + c
    row0 = wid * T_BATCH

    slot = {
        0: (srcv_0, dstv_0, dsc_0, av_0, bv_0, p_0, rows_0,
            sem_i_0, sem_g_0, sem_s_0),
        1: (srcv_1, dstv_1, dsc_1, av_1, bv_1, p_1, rows_1,
            sem_i_1, sem_g_1, sem_s_1),
        2: (srcv_2, dstv_2, dsc_2, av_2, bv_2, p_2, rows_2,
            sem_i_2, sem_g_2, sem_s_2),
        3: (srcv_3, dstv_3, dsc_3, av_3, bv_3, p_3, rows_3,
            sem_i_3, sem_g_3, sem_s_3),
    }

    def fire_idx(t, x):
        srcv, dstv = slot[x][0], slot[x][1]
        sem_i = slot[x][7]
        off = (row0 + t) * B
        pltpu.async_copy(ei_hbm.at[pl.ds(off, B)], srcv, sem_i)
        pltpu.async_copy(ei_hbm.at[pl.ds(E_EDGES + off, B)], dstv, sem_i)

    def drain_scatter(x):
        _, _, dsc, _, _, p_v, rows, _, _, sem_s = slot[x]
        pltpu.make_async_copy(rows, u_sh.at[dsc], sem_s).wait()
        pltpu.make_async_copy(p_v, s_sh.at[dsc], sem_s).wait()

    def fire_gathers(x):
        srcv, dstv, _, av, bv, _, rows, sem_i, sem_g, _ = slot[x]
        pltpu.make_async_copy(ei_hbm.at[pl.ds(0, B)], srcv, sem_i).wait()
        pltpu.make_async_copy(ei_hbm.at[pl.ds(0, B)], dstv, sem_i).wait()
        pltpu.async_copy(asrc_hbm.at[srcv], av, sem_g)
        pltpu.async_copy(adst_hbm.at[dstv], bv, sem_g)
        pltpu.async_copy(h_hbm.at[srcv], rows, sem_g)

    def process(t, x, do_i_fire):
        srcv, dstv, dsc, av, bv, p_v, rows, sem_i, sem_g, sem_s = slot[x]
        pltpu.make_async_copy(asrc_hbm.at[srcv], av, sem_g).wait()
        pltpu.make_async_copy(adst_hbm.at[dstv], bv, sem_g).wait()
        pltpu.make_async_copy(h_hbm.at[srcv], rows, sem_g).wait()
        for j in range(NG):
            sl = pl.ds(j * 16, 16)
            dsc[sl] = dstv[sl]
        if do_i_fire:
            fire_idx(t + 4, x)
        for j in range(NG):
            sl = pl.ds(j * 16, 16)
            a = av[sl] + bv[sl]
            e = jnp.where(a >= 0.0, a, 0.2 * a)
            p_v[sl] = jnp.exp(e)

        @plsc.parallel_loop(0, NG, unroll=1)
        def scale_group(g):
            pg = p_v[pl.ds(g * 16, 16)]
            for l in range(16):
                pv = pg[l]
                row = g * 16 + l
                for j in range(8):
                    sl = pl.ds(j * 16, 16)
                    rows[row, sl] = rows[row, sl] * pv

        pltpu.async_copy(rows, u_sh.at[dsc], sem_s, add=True)
        pltpu.async_copy(p_v, s_sh.at[dsc], sem_s, add=True)

    def zero_rows(b, carry):
        for j in range(8):
            rows_0[b, pl.ds(j * 16, 16)] = jnp.zeros((16,), jnp.float32)
        return carry
    lax.fori_loop(0, B, zero_rows, 0)
    for k in range(STRIPE // B):
        pltpu.sync_copy(rows_0, u_sh.at[pl.ds(s * STRIPE + k * B, B)])
        pltpu.sync_copy(rows_0.at[0, pl.ds(0, B)],
                        s_sh.at[pl.ds(s * STRIPE + k * B, B)])
    plsc.subcore_barrier()

    def phase(t, x, do_s_drain, do_g_fire, do_i_fire):
        if do_s_drain:
            drain_scatter((x + 2) % 4)
        if do_g_fire:
            fire_gathers((x + 2) % 4)
        process(t, x, do_i_fire)

    fire_idx(0, 0)
    fire_idx(1, 1)
    fire_idx(2, 2)
    fire_idx(3, 3)
    fire_gathers(0)
    fire_gathers(1)

    phase(0, 0, False, True, True)
    phase(1, 1, False, True, True)
    phase(2, 2, True, True, True)
    phase(3, 3, True, True, True)

    def body(i, carry):
        t = 4 * i
        phase(t + 0, 0, True, True, True)
        phase(t + 1, 1, True, True, True)
        phase(t + 2, 2, True, True, True)
        phase(t + 3, 3, True, True, True)
        return carry
    lax.fori_loop(1, 30, body, 0)

    phase(120, 0, True, True, True)
    phase(121, 1, True, True, False)
    phase(122, 2, True, True, False)
    phase(123, 3, True, False, False)
    phase(124, 0, True, False, False)
    drain_scatter(3)
    drain_scatter(0)

    plsc.subcore_barrier()
    pltpu.sync_copy(u_sh.at[pl.ds(s * STRIPE, STRIPE)],
                    u_out.at[c, pl.ds(s * STRIPE, STRIPE)])
    pltpu.sync_copy(s_sh.at[pl.ds(s * STRIPE, STRIPE)],
                    s_out.at[c, pl.ds(s * STRIPE, STRIPE)])


def _tc_epilogue(u_ref, s_ref, h_ref, as_ref, ad_ref, b_ref, o_ref):
    a = as_ref[...] + ad_ref[...]
    e = jnp.where(a >= 0.0, a, 0.2 * a)
    p_self = jnp.exp(e)
    u = u_ref[0, :N_NODES, :] + u_ref[1, :N_NODES, :] + p_self * h_ref[...]
    den = s_ref[0, :N_NODES, :] + s_ref[1, :N_NODES, :] + p_self + 1e-16
    o = u / den + b_ref[...][None, :]
    o_ref[...] = jnp.where(o >= 0.0, o, 0.01 * o)


@jax.jit
def _run(ei, x, W, att_src, att_dst, bias):
    h, a1, a2, asum, adsum = pl.pallas_call(
        _tc_prologue,
        out_shape=[
            jax.ShapeDtypeStruct((N_NODES, D), jnp.float32),
            jax.ShapeDtypeStruct((N_NODES,), jnp.float32),
            jax.ShapeDtypeStruct((N_NODES,), jnp.float32),
            jax.ShapeDtypeStruct((N_NODES, 1), jnp.float32),
            jax.ShapeDtypeStruct((N_NODES, 1), jnp.float32),
        ],
    )(x, W, att_src, att_dst)

    mesh = plsc.VectorSubcoreMesh(core_axis_name="c", subcore_axis_name="s")
    sc = pl.kernel(
        _sc_edges,
        out_type=[
            jax.ShapeDtypeStruct((2, N_PAD, D), jnp.float32),
            jax.ShapeDtypeStruct((2, N_PAD), jnp.float32),
        ],
        mesh=mesh,
        scratch_types=(
            [pltpu.VMEM((B,), jnp.int32) for _ in range(12)]
            + [pltpu.VMEM((B,), jnp.float32) for _ in range(12)]
            + [pltpu.VMEM((B, D), jnp.float32) for _ in range(4)]
            + [pltpu.SemaphoreType.DMA for _ in range(12)]
            + [pltpu.VMEM_SHARED((N_PAD, D), jnp.float32),
               pltpu.VMEM_SHARED((N_PAD,), jnp.float32)]
        ),
    )
    u2, s2 = sc(ei, h, a1, a2)

    out = pl.pallas_call(
        _tc_epilogue,
        out_shape=jax.ShapeDtypeStruct((N_NODES, D), jnp.float32),
    )(u2, s2.reshape(2, N_PAD, 1), h, asum, adsum, bias)
    return out


def kernel(edge_index, x, W, att_src, att_dst, bias):
    return _run(edge_index.astype(jnp.int32).reshape(-1),
                x, W, att_src, att_dst, bias)


# speedup vs baseline: 1.2911x; 1.0718x over previous
import jax
import jax.numpy as jnp
from jax import lax
from jax.experimental import pallas as pl
from jax.experimental.pallas import tpu as pltpu
import jax.experimental.pallas.tpu_sc as plsc

N_NODES = 10000
D = 128
E_EDGES = 320000
NW = 32
B = 80
NG = B // 16
T_BATCH = 125
PER_W = B * T_BATCH
N_PAD = 10240
STRIPE = N_PAD // 16


def _tc_prologue(x_ref, w_ref, as_ref, ad_ref,
                 h_ref, a1_ref, a2_ref, asum_ref, adsum_ref):
    h = jnp.dot(x_ref[...], w_ref[...], preferred_element_type=jnp.float32)
    h_ref[...] = h
    asum = jnp.sum(h * as_ref[...][None, :], axis=1, keepdims=True)
    adsum = jnp.sum(h * ad_ref[...][None, :], axis=1, keepdims=True)
    asum_ref[...] = asum
    adsum_ref[...] = adsum
    a1_ref[...] = asum.reshape(-1)
    a2_ref[...] = adsum.reshape(-1)


def _sc_edges(ei_hbm, h_hbm, asrc_hbm, adst_hbm,
              u_out, s_out,
              srcv_0, srcv_1, srcv_2, srcv_3,
              dstv_0, dstv_1, dstv_2, dstv_3,
              dsc_0, dsc_1, dsc_2, dsc_3,
              av_0, av_1, av_2, av_3, bv_0, bv_1, bv_2, bv_3,
              p_0, p_1, p_2, p_3, rows_0, rows_1, rows_2, rows_3,
              sem_i_0, sem_i_1, sem_i_2, sem_i_3,
              sem_g_0, sem_g_1, sem_g_2, sem_g_3,
              sem_s_0, sem_s_1, sem_s_2, sem_s_3,
              u_sh, s_sh):
    c = lax.axis_index("c")
    s = lax.axis_index("s")
    wid = s * 2 + c
    row0 = wid * T_BATCH

    slot = {
        0: (srcv_0, dstv_0, dsc_0, av_0, bv_0, p_0, rows_0,
            sem_i_0, sem_g_0, sem_s_0),
        1: (srcv_1, dstv_1, dsc_1, av_1, bv_1, p_1, rows_1,
            sem_i_1, sem_g_1, sem_s_1),
        2: (srcv_2, dstv_2, dsc_2, av_2, bv_2, p_2, rows_2,
            sem_i_2, sem_g_2, sem_s_2),
        3: (srcv_3, dstv_3, dsc_3, av_3, bv_3, p_3, rows_3,
            sem_i_3, sem_g_3, sem_s_3),
    }

    def fire_idx(t, x):
        srcv, dstv = slot[x][0], slot[x][1]
        sem_i = slot[x][7]
        off = (row0 + t) * B
        pltpu.async_copy(ei_hbm.at[pl.ds(off, B)], srcv, sem_i)
        pltpu.async_copy(ei_hbm.at[pl.ds(E_EDGES + off, B)], dstv, sem_i)

    def drain_scatter(x):
        _, _, dsc, _, _, p_v, rows, _, _, sem_s = slot[x]
        pltpu.make_async_copy(rows, u_sh.at[dsc], sem_s).wait()
        pltpu.make_async_copy(p_v, s_sh.at[dsc], sem_s).wait()

    def fire_gathers(x):
        srcv, dstv, _, av, bv, _, rows, sem_i, sem_g, _ = slot[x]
        pltpu.make_async_copy(ei_hbm.at[pl.ds(0, B)], srcv, sem_i).wait()
        pltpu.make_async_copy(ei_hbm.at[pl.ds(0, B)], dstv, sem_i).wait()
        pltpu.async_copy(asrc_hbm.at[srcv], av, sem_g)
        pltpu.async_copy(adst_hbm.at[dstv], bv, sem_g)
        pltpu.async_copy(h_hbm.at[srcv], rows, sem_g)

    def process(t, x, do_i_fire):
        srcv, dstv, dsc, av, bv, p_v, rows, sem_i, sem_g, sem_s = slot[x]
        pltpu.make_async_copy(asrc_hbm.at[srcv], av, sem_g).wait()
        pltpu.make_async_copy(adst_hbm.at[dstv], bv, sem_g).wait()
        pltpu.make_async_copy(h_hbm.at[srcv], rows, sem_g).wait()
        for j in range(NG):
            sl = pl.ds(j * 16, 16)
            dsc[sl] = dstv[sl]
        if do_i_fire:
            fire_idx(t + 4, x)
        for j in range(NG):
            sl = pl.ds(j * 16, 16)
            a = av[sl] + bv[sl]
            e = jnp.where(a >= 0.0, a, 0.2 * a)
            p_v[sl] = jnp.exp(e)

        @plsc.parallel_loop(0, NG, unroll=2)
        def scale_group(g):
            pg = p_v[pl.ds(g * 16, 16)]
            for l in range(16):
                pv = pg[l]
                row = g * 16 + l
                for j in range(8):
                    sl = pl.ds(j * 16, 16)
                    rows[row, sl] = rows[row, sl] * pv

        pltpu.async_copy(rows, u_sh.at[dsc], sem_s, add=True)
        pltpu.async_copy(p_v, s_sh.at[dsc], sem_s, add=True)

    def zero_rows(b, carry):
        for j in range(8):
            rows_0[b, pl.ds(j * 16, 16)] = jnp.zeros((16,), jnp.float32)
        return carry
    lax.fori_loop(0, B, zero_rows, 0)
    for k in range(STRIPE // B):
        pltpu.sync_copy(rows_0, u_sh.at[pl.ds(s * STRIPE + k * B, B)])
        pltpu.sync_copy(rows_0.at[0, pl.ds(0, B)],
                        s_sh.at[pl.ds(s * STRIPE + k * B, B)])
    plsc.subcore_barrier()

    def phase(t, x, do_s_drain, do_g_fire, do_i_fire):
        if do_s_drain:
            drain_scatter((x + 2) % 4)
        if do_g_fire:
            fire_gathers((x + 2) % 4)
        process(t, x, do_i_fire)

    fire_idx(0, 0)
    fire_idx(1, 1)
    fire_idx(2, 2)
    fire_idx(3, 3)
    fire_gathers(0)
    fire_gathers(1)

    phase(0, 0, False, True, True)
    phase(1, 1, False, True, True)
    phase(2, 2, True, True, True)
    phase(3, 3, True, True, True)

    def body(i, carry):
        t = 4 * i
        phase(t + 0, 0, True, True, True)
        phase(t + 1, 1, True, True, True)
        phase(t + 2, 2, True, True, True)
        phase(t + 3, 3, True, True, True)
        return carry
    lax.fori_loop(1, 30, body, 0)

    phase(120, 0, True, True, True)
    phase(121, 1, True, True, False)
    phase(122, 2, True, True, False)
    phase(123, 3, True, False, False)
    phase(124, 0, True, False, False)
    drain_scatter(3)
    drain_scatter(0)

    plsc.subcore_barrier()
    pltpu.sync_copy(u_sh.at[pl.ds(s * STRIPE, STRIPE)],
                    u_out.at[c, pl.ds(s * STRIPE, STRIPE)])
    pltpu.sync_copy(s_sh.at[pl.ds(s * STRIPE, STRIPE)],
                    s_out.at[c, pl.ds(s * STRIPE, STRIPE)])


def _tc_epilogue(u_ref, s_ref, h_ref, as_ref, ad_ref, b_ref, o_ref):
    a = as_ref[...] + ad_ref[...]
    e = jnp.where(a >= 0.0, a, 0.2 * a)
    p_self = jnp.exp(e)
    u = u_ref[0, :N_NODES, :] + u_ref[1, :N_NODES, :] + p_self * h_ref[...]
    den = s_ref[0, :N_NODES, :] + s_ref[1, :N_NODES, :] + p_self + 1e-16
    o = u / den + b_ref[...][None, :]
    o_ref[...] = jnp.where(o >= 0.0, o, 0.01 * o)


@jax.jit
def _run(ei, x, W, att_src, att_dst, bias):
    h, a1, a2, asum, adsum = pl.pallas_call(
        _tc_prologue,
        out_shape=[
            jax.ShapeDtypeStruct((N_NODES, D), jnp.float32),
            jax.ShapeDtypeStruct((N_NODES,), jnp.float32),
            jax.ShapeDtypeStruct((N_NODES,), jnp.float32),
            jax.ShapeDtypeStruct((N_NODES, 1), jnp.float32),
            jax.ShapeDtypeStruct((N_NODES, 1), jnp.float32),
        ],
    )(x, W, att_src, att_dst)

    mesh = plsc.VectorSubcoreMesh(core_axis_name="c", subcore_axis_name="s")
    sc = pl.kernel(
        _sc_edges,
        out_type=[
            jax.ShapeDtypeStruct((2, N_PAD, D), jnp.float32),
            jax.ShapeDtypeStruct((2, N_PAD), jnp.float32),
        ],
        mesh=mesh,
        scratch_types=(
            [pltpu.VMEM((B,), jnp.int32) for _ in range(12)]
            + [pltpu.VMEM((B,), jnp.float32) for _ in range(12)]
            + [pltpu.VMEM((B, D), jnp.float32) for _ in range(4)]
            + [pltpu.SemaphoreType.DMA for _ in range(12)]
            + [pltpu.VMEM_SHARED((N_PAD, D), jnp.float32),
               pltpu.VMEM_SHARED((N_PAD,), jnp.float32)]
        ),
    )
    u2, s2 = sc(ei, h, a1, a2)

    out = pl.pallas_call(
        _tc_epilogue,
        out_shape=jax.ShapeDtypeStruct((N_NODES, D), jnp.float32),
    )(u2, s2.reshape(2, N_PAD, 1), h, asum, adsum, bias)
    return out


def kernel(edge_index, x, W, att_src, att_dst, bias):
    return _run(edge_index.astype(jnp.int32).reshape(-1),
                x, W, att_src, att_dst, bias)
